# Initial kernel scaffold; baseline (speedup 1.0000x reference)
import jax
import jax.numpy as jnp
from jax.experimental import pallas as pl


def kernel(x, Wg, bg, We, be):
    raise NotImplementedError("write your pallas kernel here")


# speedup vs baseline: 6.8524x; 6.8524x over previous
import jax
import jax.numpy as jnp
from jax.experimental import pallas as pl
from jax.experimental.pallas import tpu as pltpu

B = 2048
IN = 1024
OUT = 1024
E = 8


def _moe_body(x_ref, wg_ref, bg_ref, be_ref, we_ref, out_ref, s_ref, acc_ref):
    e = pl.program_id(0)

    @pl.when(e == 0)
    def _init():
        xx = x_ref[...]
        logits = jax.lax.dot_general(
            xx, wg_ref[...], (((1,), (1,)), ((), ())),
            preferred_element_type=jnp.float32) + bg_ref[...]
        iota = jax.lax.broadcasted_iota(jnp.int32, (B, E), 1)
        v1 = jnp.max(logits, axis=1, keepdims=True)
        i1 = jnp.min(jnp.where(logits == v1, iota, E + 1), axis=1,
                     keepdims=True)
        masked = jnp.where(iota == i1, -jnp.inf, logits)
        v2 = jnp.max(masked, axis=1, keepdims=True)
        i2 = jnp.min(jnp.where(masked == v2, iota, E + 1), axis=1,
                     keepdims=True)
        t = jnp.exp(v2 - v1)
        w1 = 1.0 / (1.0 + t)
        w2 = t / (1.0 + t)
        g = jnp.where(iota == i1, w1, 0.0) + jnp.where(iota == i2, w2, 0.0)
        s_ref[...] = jax.lax.dot_general(
            g, xx, (((0,), (0,)), ((), ())),
            preferred_element_type=jnp.float32)
        c = jnp.sum(g, axis=0, keepdims=True)
        acc_ref[...] = jax.lax.dot_general(
            c, be_ref[...], (((1,), (0,)), ((), ())),
            preferred_element_type=jnp.float32)

    se = s_ref[pl.ds(e, 1), :]
    acc_ref[...] += jax.lax.dot_general(
        se, we_ref[0], (((1,), (1,)), ((), ())),
        preferred_element_type=jnp.float32)

    @pl.when(e == E - 1)
    def _emit():
        out_ref[...] = jnp.broadcast_to(acc_ref[...], (B, OUT))


def kernel(x, Wg, bg, We, be):
    bg2 = bg.reshape(1, E)
    return pl.pallas_call(
        _moe_body,
        grid=(E,),
        in_specs=[
            pl.BlockSpec((B, IN), lambda e: (0, 0)),
            pl.BlockSpec((E, IN), lambda e: (0, 0)),
            pl.BlockSpec((1, E), lambda e: (0, 0)),
            pl.BlockSpec((E, OUT), lambda e: (0, 0)),
            pl.BlockSpec((1, OUT, IN), lambda e: (e, 0, 0)),
        ],
        out_specs=pl.BlockSpec((B, OUT), lambda e: (0, 0)),
        out_shape=jax.ShapeDtypeStruct((B, OUT), jnp.float32),
        scratch_shapes=[
            pltpu.VMEM((E, IN), jnp.float32),
            pltpu.VMEM((1, OUT), jnp.float32),
        ],
    )(x, Wg, bg2, be, We)
